# baseline (speedup 1.0000x reference)
import jax
import jax.numpy as jnp
from jax.experimental import pallas as pl
from jax.experimental.pallas import tpu as pltpu
from jax.experimental.pallas import tpu_sc as plsc

_TM = 512
_RB = 128
_KC = 1024
_NCHUNK = 2
_K = 8192
_D = 256
_GATHER_WIN = 128


def _argmin_body(x_ref, xsq_ref, embT_ref, esq_ref, fio_ref, ind_ref):
    esq = esq_ref[...]
    lane = fio_ref[0:1, 0:128]
    for r0 in range(0, _TM, _RB):
        rows = slice(r0, r0 + _RB)
        x2_r = x_ref[rows, :] + x_ref[rows, :]
        xsq_r = xsq_ref[rows, :]
        M = None
        G = None
        for c0 in range(0, _K, _KC):
            xe = jax.lax.dot_general(
                x2_r, embT_ref[:, c0:c0 + _KC],
                dimension_numbers=(((1,), (0,)), ((), ())),
                preferred_element_type=jnp.float32)
            for g0 in range(0, _KC, 128):
                t = (xsq_r - xe[:, g0:g0 + 128]) + esq[:, c0 + g0:c0 + g0 + 128]
                if M is None:
                    M = t
                    G = jnp.zeros((_RB, 128), jnp.float32)
                else:
                    lt = t < M
                    M = jnp.where(lt, t, M)
                    G = jnp.where(lt, jnp.float32((c0 + g0) // 128), G)
        k_idx = G * 128.0 + lane
        m = jnp.min(M, axis=1, keepdims=True)
        cand = jnp.where(M == m, k_idx, jnp.float32(3.0e38))
        arg = jnp.min(cand, axis=1)
        ind_ref[0, 0, r0:r0 + _RB] = arg.astype(jnp.int32)


def _compute_indices(xf, xsq, embT, esq, fio, row0, nrows):
    grid = nrows // _TM
    t0 = row0 // _TM
    out = pl.pallas_call(
        _argmin_body,
        grid=(grid,),
        in_specs=[
            pl.BlockSpec((_TM, _D), lambda i: (t0 + i, 0)),
            pl.BlockSpec((_TM, 1), lambda i: (t0 + i, 0)),
            pl.BlockSpec((_D, _K), lambda i: (0, 0)),
            pl.BlockSpec((1, _K), lambda i: (0, 0)),
            pl.BlockSpec((1, _K), lambda i: (0, 0)),
        ],
        out_specs=pl.BlockSpec((1, 1, _TM), lambda i: (i, 0, 0)),
        out_shape=jax.ShapeDtypeStruct((grid, 1, _TM), jnp.int32),
    )(xf, xsq, embT, esq, fio)
    return out.reshape(nrows)


def _gather_rows(table, idx):
    n = idx.shape[0]
    d = table.shape[1]
    idx2 = idx.reshape(1, n)
    mesh = plsc.VectorSubcoreMesh(core_axis_name="core",
                                  subcore_axis_name="subcore")

    @pl.kernel(out_type=jax.ShapeDtypeStruct((n, d), table.dtype), mesh=mesh)
    def k(tab_hbm, i_hbm, o_hbm):
        def body(i_vmem, o_vmem):
            pltpu.sync_copy(tab_hbm.at[i_vmem.at[0]], o_vmem)

        pltpu.emit_pipeline(
            body,
            grid=(n // _GATHER_WIN,),
            in_specs=[pl.BlockSpec((1, _GATHER_WIN), index_map=lambda i: (0, i))],
            out_specs=[pl.BlockSpec((_GATHER_WIN, d), index_map=lambda i: (i, 0))],
            core_axis_name=("core", "subcore"),
            dimension_semantics=(pltpu.PARALLEL,),
        )(i_hbm, o_hbm)

    return k(table, idx2)


def kernel(x, embeddings):
    x = x.astype(jnp.float32)
    xf = x.reshape(-1, x.shape[-1])
    emb = embeddings[0]
    embT = emb.T.astype(jnp.bfloat16)
    xsq = jnp.sum(xf ** 2, axis=-1, keepdims=True)
    esq = jnp.sum(embeddings ** 2, axis=-1)
    fio = jnp.arange(_K, dtype=jnp.float32)[None, :]
    n = xf.shape[0]
    cs = n // _NCHUNK
    inds, qs = [], []
    for c in range(_NCHUNK):
        ind_c = _compute_indices(xf, xsq, embT, esq, fio, c * cs, cs)
        qs.append(_gather_rows(emb, ind_c))
        inds.append(ind_c)
    q = jnp.concatenate(qs, axis=0)
    ind = jnp.concatenate(inds, axis=0)
    return q.reshape(x.shape), ind.reshape(x.shape[:-1])


# speedup vs baseline: 1.0586x; 1.0586x over previous
import jax
import jax.numpy as jnp
from jax.experimental import pallas as pl
from jax.experimental.pallas import tpu as pltpu
from jax.experimental.pallas import tpu_sc as plsc

_TM = 512
_RB = 128
_KC = 1024
_NCHUNK = 1
_K = 8192
_D = 256
_GATHER_WIN = 128


def _argmin_body(x_ref, xsq_ref, embT_ref, esq_ref, ind_ref):
    esq = esq_ref[...]
    lane = jax.lax.broadcasted_iota(jnp.int32, (1, 128), 1).astype(jnp.float32)
    for r0 in range(0, _TM, _RB):
        rows = slice(r0, r0 + _RB)
        x2_r = x_ref[rows, :] + x_ref[rows, :]
        xsq_r = xsq_ref[rows, :]
        M = None
        G = None
        for c0 in range(0, _K, _KC):
            xe = jax.lax.dot_general(
                x2_r, embT_ref[:, c0:c0 + _KC],
                dimension_numbers=(((1,), (0,)), ((), ())),
                preferred_element_type=jnp.float32)
            for g0 in range(0, _KC, 128):
                t = (xsq_r - xe[:, g0:g0 + 128]) + esq[:, c0 + g0:c0 + g0 + 128]
                if M is None:
                    M = t
                    G = jnp.zeros((_RB, 128), jnp.float32)
                else:
                    lt = t < M
                    M = jnp.where(lt, t, M)
                    G = jnp.where(lt, jnp.float32((c0 + g0) // 128), G)
        k_idx = G * 128.0 + lane
        m = jnp.min(M, axis=1, keepdims=True)
        cand = jnp.where(M == m, k_idx, jnp.float32(3.0e38))
        arg = jnp.min(cand, axis=1)
        ind_ref[0, 0, r0:r0 + _RB] = arg.astype(jnp.int32)


def _compute_indices(xf, xsq, embT, esq, row0, nrows):
    grid = nrows // _TM
    t0 = row0 // _TM
    out = pl.pallas_call(
        _argmin_body,
        grid=(grid,),
        in_specs=[
            pl.BlockSpec((_TM, _D), lambda i: (t0 + i, 0)),
            pl.BlockSpec((_TM, 1), lambda i: (t0 + i, 0)),
            pl.BlockSpec((_D, _K), lambda i: (0, 0)),
            pl.BlockSpec((1, _K), lambda i: (0, 0)),
        ],
        out_specs=pl.BlockSpec((1, 1, _TM), lambda i: (i, 0, 0)),
        out_shape=jax.ShapeDtypeStruct((grid, 1, _TM), jnp.int32),
    )(xf, xsq, embT, esq)
    return out.reshape(nrows)


def _gather_rows(table, idx):
    n = idx.shape[0]
    d = table.shape[1]
    idx2 = idx.reshape(1, n)
    mesh = plsc.VectorSubcoreMesh(core_axis_name="core",
                                  subcore_axis_name="subcore")

    @pl.kernel(out_type=jax.ShapeDtypeStruct((n, d), table.dtype), mesh=mesh)
    def k(tab_hbm, i_hbm, o_hbm):
        def body(i_vmem, o_vmem):
            pltpu.sync_copy(tab_hbm.at[i_vmem.at[0]], o_vmem)

        pltpu.emit_pipeline(
            body,
            grid=(n // _GATHER_WIN,),
            in_specs=[pl.BlockSpec((1, _GATHER_WIN), index_map=lambda i: (0, i))],
            out_specs=[pl.BlockSpec((_GATHER_WIN, d), index_map=lambda i: (i, 0))],
            core_axis_name=("core", "subcore"),
            dimension_semantics=(pltpu.PARALLEL,),
        )(i_hbm, o_hbm)

    return k(table, idx2)


def kernel(x, embeddings):
    x = x.astype(jnp.float32)
    xf = x.reshape(-1, x.shape[-1])
    emb = embeddings[0]
    embT = emb.T.astype(jnp.bfloat16)
    xsq = jnp.sum(xf ** 2, axis=-1, keepdims=True)
    esq = jnp.sum(embeddings ** 2, axis=-1)
    n = xf.shape[0]
    cs = n // _NCHUNK
    inds, qs = [], []
    for c in range(_NCHUNK):
        ind_c = _compute_indices(xf, xsq, embT, esq, c * cs, cs)
        qs.append(_gather_rows(emb, ind_c))
        inds.append(ind_c)
    q = qs[0] if _NCHUNK == 1 else jnp.concatenate(qs, axis=0)
    ind = inds[0] if _NCHUNK == 1 else jnp.concatenate(inds, axis=0)
    return q.reshape(x.shape), ind.reshape(x.shape[:-1])


# speedup vs baseline: 1.1547x; 1.0907x over previous
import jax
import jax.numpy as jnp
from jax.experimental import pallas as pl
from jax.experimental.pallas import tpu as pltpu
from jax.experimental.pallas import tpu_sc as plsc

_TM = 512
_RB = 128
_KC = 1024
_NCHUNK = 1
_K = 8192
_D = 256
_GATHER_WIN = 128


def _argmin_body(x_ref, embT_ref, esq_ref, ind_ref):
    esq = esq_ref[...]
    lane = jax.lax.broadcasted_iota(jnp.int32, (1, 128), 1).astype(jnp.float32)
    for r0 in range(0, _TM, _RB):
        rows = slice(r0, r0 + _RB)
        x_r = x_ref[rows, :]
        x2_r = x_r + x_r
        xsq_r = jnp.sum(x_r * x_r, axis=1, keepdims=True)
        M = None
        G = None
        for c0 in range(0, _K, _KC):
            xe = jax.lax.dot_general(
                x2_r, embT_ref[:, c0:c0 + _KC],
                dimension_numbers=(((1,), (0,)), ((), ())),
                preferred_element_type=jnp.float32)
            for g0 in range(0, _KC, 128):
                t = (xsq_r - xe[:, g0:g0 + 128]) + esq[:, c0 + g0:c0 + g0 + 128]
                if M is None:
                    M = t
                    G = jnp.zeros((_RB, 128), jnp.float32)
                else:
                    lt = t < M
                    M = jnp.where(lt, t, M)
                    G = jnp.where(lt, jnp.float32((c0 + g0) // 128), G)
        k_idx = G * 128.0 + lane
        m = jnp.min(M, axis=1, keepdims=True)
        cand = jnp.where(M == m, k_idx, jnp.float32(3.0e38))
        arg = jnp.min(cand, axis=1)
        ind_ref[0, 0, r0:r0 + _RB] = arg.astype(jnp.int32)


def _compute_indices(xf, embT, esq, row0, nrows):
    grid = nrows // _TM
    t0 = row0 // _TM
    out = pl.pallas_call(
        _argmin_body,
        grid=(grid,),
        in_specs=[
            pl.BlockSpec((_TM, _D), lambda i: (t0 + i, 0)),
            pl.BlockSpec((_D, _K), lambda i: (0, 0)),
            pl.BlockSpec((1, _K), lambda i: (0, 0)),
        ],
        out_specs=pl.BlockSpec((1, 1, _TM), lambda i: (i, 0, 0)),
        out_shape=jax.ShapeDtypeStruct((grid, 1, _TM), jnp.int32),
    )(xf, embT, esq)
    return out.reshape(nrows)


def _gather_rows(table, idx):
    n = idx.shape[0]
    d = table.shape[1]
    idx2 = idx.reshape(1, n)
    mesh = plsc.VectorSubcoreMesh(core_axis_name="core",
                                  subcore_axis_name="subcore")

    @pl.kernel(out_type=jax.ShapeDtypeStruct((n, d), table.dtype), mesh=mesh)
    def k(tab_hbm, i_hbm, o_hbm):
        def body(i_vmem, o_vmem):
            pltpu.sync_copy(tab_hbm.at[i_vmem.at[0]], o_vmem)

        pltpu.emit_pipeline(
            body,
            grid=(n // _GATHER_WIN,),
            in_specs=[pl.BlockSpec((1, _GATHER_WIN), index_map=lambda i: (0, i))],
            out_specs=[pl.BlockSpec((_GATHER_WIN, d), index_map=lambda i: (i, 0))],
            core_axis_name=("core", "subcore"),
            dimension_semantics=(pltpu.PARALLEL,),
        )(i_hbm, o_hbm)

    return k(table, idx2)


def kernel(x, embeddings):
    x = x.astype(jnp.float32)
    xf = x.reshape(-1, x.shape[-1])
    emb = embeddings[0]
    embT = emb.T.astype(jnp.bfloat16)
    esq = jnp.sum(embeddings ** 2, axis=-1)
    n = xf.shape[0]
    cs = n // _NCHUNK
    inds, qs = [], []
    for c in range(_NCHUNK):
        ind_c = _compute_indices(xf, embT, esq, c * cs, cs)
        qs.append(_gather_rows(emb, ind_c))
        inds.append(ind_c)
    q = qs[0] if _NCHUNK == 1 else jnp.concatenate(qs, axis=0)
    ind = inds[0] if _NCHUNK == 1 else jnp.concatenate(inds, axis=0)
    return q.reshape(x.shape), ind.reshape(x.shape[:-1])


# speedup vs baseline: 1.1725x; 1.0154x over previous
import jax
import jax.numpy as jnp
from jax.experimental import pallas as pl
from jax.experimental.pallas import tpu as pltpu
from jax.experimental.pallas import tpu_sc as plsc

_TM = 1024
_RB = 128
_KC = 1024
_NCHUNK = 1
_K = 8192
_D = 256
_GATHER_WIN = 128


def _argmin_body(x_ref, embT_ref, esq_ref, ind_ref):
    esq = esq_ref[...]
    lane = jax.lax.broadcasted_iota(jnp.int32, (1, 128), 1).astype(jnp.float32)
    for r0 in range(0, _TM, _RB):
        rows = slice(r0, r0 + _RB)
        x_r = x_ref[rows, :]
        x2_r = x_r + x_r
        xsq_r = jnp.sum(x_r * x_r, axis=1, keepdims=True)
        M = None
        G = None
        for c0 in range(0, _K, _KC):
            xe = jax.lax.dot_general(
                x2_r, embT_ref[:, c0:c0 + _KC],
                dimension_numbers=(((1,), (0,)), ((), ())),
                preferred_element_type=jnp.float32)
            for g0 in range(0, _KC, 128):
                t = (xsq_r - xe[:, g0:g0 + 128]) + esq[:, c0 + g0:c0 + g0 + 128]
                if M is None:
                    M = t
                    G = jnp.zeros((_RB, 128), jnp.float32)
                else:
                    lt = t < M
                    M = jnp.where(lt, t, M)
                    G = jnp.where(lt, jnp.float32((c0 + g0) // 128), G)
        k_idx = G * 128.0 + lane
        m = jnp.min(M, axis=1, keepdims=True)
        cand = jnp.where(M == m, k_idx, jnp.float32(3.0e38))
        arg = jnp.min(cand, axis=1)
        ind_ref[0, 0, r0:r0 + _RB] = arg.astype(jnp.int32)


def _compute_indices(xf, embT, esq, row0, nrows):
    grid = nrows // _TM
    t0 = row0 // _TM
    out = pl.pallas_call(
        _argmin_body,
        grid=(grid,),
        in_specs=[
            pl.BlockSpec((_TM, _D), lambda i: (t0 + i, 0)),
            pl.BlockSpec((_D, _K), lambda i: (0, 0)),
            pl.BlockSpec((1, _K), lambda i: (0, 0)),
        ],
        out_specs=pl.BlockSpec((1, 1, _TM), lambda i: (i, 0, 0)),
        out_shape=jax.ShapeDtypeStruct((grid, 1, _TM), jnp.int32),
    )(xf, embT, esq)
    return out.reshape(nrows)


def _gather_rows(table, idx):
    n = idx.shape[0]
    d = table.shape[1]
    idx2 = idx.reshape(1, n)
    mesh = plsc.VectorSubcoreMesh(core_axis_name="core",
                                  subcore_axis_name="subcore")

    @pl.kernel(out_type=jax.ShapeDtypeStruct((n, d), table.dtype), mesh=mesh)
    def k(tab_hbm, i_hbm, o_hbm):
        def body(i_vmem, o_vmem):
            pltpu.sync_copy(tab_hbm.at[i_vmem.at[0]], o_vmem)

        pltpu.emit_pipeline(
            body,
            grid=(n // _GATHER_WIN,),
            in_specs=[pl.BlockSpec((1, _GATHER_WIN), index_map=lambda i: (0, i))],
            out_specs=[pl.BlockSpec((_GATHER_WIN, d), index_map=lambda i: (i, 0))],
            core_axis_name=("core", "subcore"),
            dimension_semantics=(pltpu.PARALLEL,),
        )(i_hbm, o_hbm)

    return k(table, idx2)


def kernel(x, embeddings):
    x = x.astype(jnp.float32)
    xf = x.reshape(-1, x.shape[-1])
    emb = embeddings[0]
    embT = emb.T.astype(jnp.bfloat16)
    esq = jnp.sum(embeddings ** 2, axis=-1)
    n = xf.shape[0]
    cs = n // _NCHUNK
    inds, qs = [], []
    for c in range(_NCHUNK):
        ind_c = _compute_indices(xf, embT, esq, c * cs, cs)
        qs.append(_gather_rows(emb, ind_c))
        inds.append(ind_c)
    q = qs[0] if _NCHUNK == 1 else jnp.concatenate(qs, axis=0)
    ind = inds[0] if _NCHUNK == 1 else jnp.concatenate(inds, axis=0)
    return q.reshape(x.shape), ind.reshape(x.shape[:-1])


# speedup vs baseline: 1.2063x; 1.0288x over previous
import jax
import jax.numpy as jnp
from jax.experimental import pallas as pl
from jax.experimental.pallas import tpu as pltpu
from jax.experimental.pallas import tpu_sc as plsc

_TM = 1024
_RB = 128
_KC = 1024
_NCHUNK = 1
_K = 8192
_D = 256
_GATHER_WIN = 128


def _argmin_body(x_ref, emb_ref, ind_ref, embT_ref, esq_ref):
    @pl.when(pl.program_id(0) == 0)
    def _init():
        ev = emb_ref[...]
        embT_ref[...] = jnp.transpose(ev).astype(jnp.bfloat16)
        esq_ref[...] = jnp.sum(ev * ev, axis=1).reshape(1, _K)

    esq = esq_ref[...]
    lane = jax.lax.broadcasted_iota(jnp.int32, (1, 128), 1).astype(jnp.float32)
    for r0 in range(0, _TM, _RB):
        rows = slice(r0, r0 + _RB)
        x_r = x_ref[rows, :]
        x2_r = x_r + x_r
        xsq_r = jnp.sum(x_r * x_r, axis=1, keepdims=True)
        M = None
        G = None
        for c0 in range(0, _K, _KC):
            xe = jax.lax.dot_general(
                x2_r, embT_ref[:, c0:c0 + _KC],
                dimension_numbers=(((1,), (0,)), ((), ())),
                preferred_element_type=jnp.float32)
            for g0 in range(0, _KC, 128):
                t = (xsq_r - xe[:, g0:g0 + 128]) + esq[:, c0 + g0:c0 + g0 + 128]
                if M is None:
                    M = t
                    G = jnp.zeros((_RB, 128), jnp.float32)
                else:
                    lt = t < M
                    M = jnp.where(lt, t, M)
                    G = jnp.where(lt, jnp.float32((c0 + g0) // 128), G)
        k_idx = G * 128.0 + lane
        m = jnp.min(M, axis=1, keepdims=True)
        cand = jnp.where(M == m, k_idx, jnp.float32(3.0e38))
        arg = jnp.min(cand, axis=1)
        ind_ref[0, 0, r0:r0 + _RB] = arg.astype(jnp.int32)


def _compute_indices(xf, emb, row0, nrows):
    grid = nrows // _TM
    t0 = row0 // _TM
    out = pl.pallas_call(
        _argmin_body,
        grid=(grid,),
        in_specs=[
            pl.BlockSpec((_TM, _D), lambda i: (t0 + i, 0)),
            pl.BlockSpec((_K, _D), lambda i: (0, 0)),
        ],
        out_specs=pl.BlockSpec((1, 1, _TM), lambda i: (i, 0, 0)),
        out_shape=jax.ShapeDtypeStruct((grid, 1, _TM), jnp.int32),
        scratch_shapes=[
            pltpu.VMEM((_D, _K), jnp.bfloat16),
            pltpu.VMEM((1, _K), jnp.float32),
        ],
    )(xf, emb)
    return out.reshape(nrows)


def _gather_rows(table, idx):
    n = idx.shape[0]
    d = table.shape[1]
    idx2 = idx.reshape(1, n)
    mesh = plsc.VectorSubcoreMesh(core_axis_name="core",
                                  subcore_axis_name="subcore")

    @pl.kernel(out_type=jax.ShapeDtypeStruct((n, d), table.dtype), mesh=mesh)
    def k(tab_hbm, i_hbm, o_hbm):
        def body(i_vmem, o_vmem):
            pltpu.sync_copy(tab_hbm.at[i_vmem.at[0]], o_vmem)

        pltpu.emit_pipeline(
            body,
            grid=(n // _GATHER_WIN,),
            in_specs=[pl.BlockSpec((1, _GATHER_WIN), index_map=lambda i: (0, i))],
            out_specs=[pl.BlockSpec((_GATHER_WIN, d), index_map=lambda i: (i, 0))],
            core_axis_name=("core", "subcore"),
            dimension_semantics=(pltpu.PARALLEL,),
        )(i_hbm, o_hbm)

    return k(table, idx2)


def kernel(x, embeddings):
    x = x.astype(jnp.float32)
    xf = x.reshape(-1, x.shape[-1])
    emb = embeddings[0]
    n = xf.shape[0]
    cs = n // _NCHUNK
    inds, qs = [], []
    for c in range(_NCHUNK):
        ind_c = _compute_indices(xf, emb, c * cs, cs)
        qs.append(_gather_rows(emb, ind_c))
        inds.append(ind_c)
    q = qs[0] if _NCHUNK == 1 else jnp.concatenate(qs, axis=0)
    ind = inds[0] if _NCHUNK == 1 else jnp.concatenate(inds, axis=0)
    return q.reshape(x.shape), ind.reshape(x.shape[:-1])
